# baseline (speedup 1.0000x reference)
import jax
import jax.numpy as jnp
from jax import lax
from jax.experimental import pallas as pl
from jax.experimental.pallas import tpu as pltpu
from jax.experimental.pallas import tpu_sc as plsc

GRID = 32
RSTEP = 0.1
RMAX = 3.0
NATOMS = 128
NRAD = 64
L = 16
PLANE = GRID * GRID


def _splat(vec, j):
  return jnp.full((L,), vec[j], dtype=vec.dtype)


def _sc_body(ax_h, ay_h, az_h, dens_h, gv_h, out_h,
             ax_v, ay_v, az_v, dens_v, gv_v, plane_v, sem):
  cid = lax.axis_index("c")
  sid = lax.axis_index("s")
  wid = sid * 2 + cid

  pltpu.async_copy(dens_h, dens_v, sem).wait()
  pltpu.async_copy(ax_h, ax_v, sem).wait()
  pltpu.async_copy(ay_h, ay_v, sem).wait()
  pltpu.async_copy(az_h, az_v, sem).wait()
  pltpu.async_copy(gv_h, gv_v, sem).wait()

  gv = gv_v[...]
  g00 = _splat(gv, 0)
  g01 = _splat(gv, 1)
  g02 = _splat(gv, 2)
  g11 = _splat(gv, 3)
  g12 = _splat(gv, 4)
  g22 = _splat(gv, 5)
  ngd = _splat(gv, 6)
  invg11 = _splat(gv, 7)
  inv_rstep = _splat(gv, 8)
  ninvg00 = _splat(gv, 9)
  hwx = _splat(gv, 10)
  clampx = _splat(gv, 11)
  nchx = gv.astype(jnp.int32)[12]

  iota = lax.iota(jnp.int32, L)

  zero16 = jnp.zeros((L,), jnp.float32)

  def zero_body(r, _):
    plane_v[pl.ds(r * L, L)] = zero16
    return _

  lax.fori_loop(0, (PLANE + L) // L, zero_body, None)

  rmax2 = jnp.full((L,), RMAX * RMAX, jnp.float32)
  rmax2_pad = jnp.full((L,), RMAX * RMAX + 1e-3, jnp.float32)
  half = jnp.full((L,), 0.5, jnp.float32)
  three_half = jnp.full((L,), 1.5, jnp.float32)
  magic = jnp.full((L,), 0x5F3759DF, jnp.int32)
  one_i = jnp.full((L,), 1, jnp.int32)
  zero_i = jnp.full((L,), 0, jnp.int32)
  cap31 = jnp.full((L,), 31, jnp.int32)
  max_off = PLANE - L + L

  def newton_rsqrt(a):
    bits = plsc.bitcast(a, jnp.int32)
    y0 = plsc.bitcast(magic - lax.shift_right_logical(bits, 1), jnp.float32)
    hx = half * a
    y0 = y0 * (three_half - hx * y0 * y0)
    y0 = y0 * (three_half - hx * y0 * y0)
    return y0

  def half_pass(zplane, ymin, ymax):
    zfs = jnp.full((L,), zplane, dtype=jnp.int32).astype(jnp.float32)
    ymin_f = jnp.full((L,), ymin, jnp.int32).astype(jnp.float32)
    ymax_f = jnp.full((L,), ymax, jnp.int32).astype(jnp.float32)

    def chunk_body(c, _):
      base = c * L
      axv = ax_v[pl.ds(base, L)]
      ayv = ay_v[pl.ds(base, L)]
      azv = az_v[pl.ds(base, L)]

      dzv = zfs - azv
      cdzv = g22 * dzv
      remy = rmax2_pad - cdzv * cdzv
      ok = remy >= 0.0
      remy_nn = jnp.maximum(remy, 0.0)
      sy = remy_nn * newton_rsqrt(remy_nn)
      sy = sy * 1.00002 + 1e-3
      sy = jnp.where(ok, sy, -1.0)
      cyv = ngd * dzv
      hw = sy * invg11
      ylo_f = jnp.maximum(ayv + cyv - hw, ymin_f)
      yhi_f = jnp.minimum(ayv + cyv + hw, ymax_f)
      ilo = ylo_f.astype(jnp.int32)
      ylov = ilo + jnp.where(ilo.astype(jnp.float32) < ylo_f, one_i, zero_i)
      ycntv = yhi_f.astype(jnp.int32) - ylov + 1

      for j in range(L):
        ycnt_s = ycntv[j]

        @pl.when(ycnt_s > 0)
        def _():
          n = base + j
          nv = jnp.full((L,), n, jnp.int32)
          ays = _splat(ayv, j)
          dzs = _splat(dzv, j)
          cdzs = _splat(cdzv, j)
          ylo_s = ylov[j]
          cdz2 = cdzs * cdzs
          g12dz = g12 * dzs
          xk = g02 * dzs - g00 * _splat(axv, j)
          yf0 = jnp.full((L,), ylo_s, jnp.int32).astype(jnp.float32)

          def row_body(yi, yfv):
            y = ylo_s + yi
            cdy = g12dz + g11 * (yfv - ays)
            cyz2 = cdz2 + cdy * cdy
            rowbase = xk + g01 * (yfv - ays)
            cxv = rowbase * ninvg00
            xlo_f = jnp.minimum(jnp.maximum(cxv - hwx, 0.0), clampx)
            xlo_s = xlo_f.astype(jnp.int32)[0]
            y32 = y * GRID

            def x_body(k, _c):
              xs = xlo_s + k * L
              xv_i = jnp.full((L,), xs, jnp.int32) + iota
              xf = xv_i.astype(jnp.float32)
              cdx = rowbase + g00 * xf
              d2 = cdx * cdx + cyz2
              m = jnp.logical_and(d2 <= rmax2, xv_i <= cap31)
              y0 = newton_rsqrt(d2)
              rad = (d2 * y0) * inv_rstep
              rad = jnp.minimum(rad, 62.99)
              il = rad.astype(jnp.int32)
              wh = rad - il.astype(jnp.float32)
              dl = plsc.load_gather(dens_v, [nv, il])
              dh = plsc.load_gather(dens_v, [nv, il + 1])
              dens = dl + wh * (dh - dl)
              contrib = jnp.where(m, dens, 0.0)
              off = jnp.minimum(y32 + xs, max_off)
              plsc.addupdate(plane_v.at[pl.ds(off, L)], contrib)
              return _c

            lax.fori_loop(0, nchx, x_body, None)
            return yfv + 1.0

          lax.fori_loop(0, ycnt_s, row_body, yf0)

      return _

    lax.fori_loop(0, NATOMS // L, chunk_body, None)

  z1 = wid
  z2 = (GRID - 1) - wid
  half_pass(z1, 0, GRID // 2 - 1)
  half_pass(z2, GRID // 2, GRID - 1)

  pltpu.async_copy(plane_v.at[pl.ds(0, PLANE // 2)],
                   out_h.at[z1, pl.ds(0, PLANE // 2)], sem).wait()
  pltpu.async_copy(plane_v.at[pl.ds(PLANE // 2, PLANE // 2)],
                   out_h.at[z2, pl.ds(PLANE // 2, PLANE // 2)], sem).wait()


def kernel(coordinates, active, occupancies, lmax, radial_densities,
           grid_to_cartesian):
  del lmax
  dtype = jnp.float32
  coords = coordinates[0].astype(dtype)
  ax = coords[:, 0]
  ay = coords[:, 1]
  az = coords[:, 2]
  occ = (occupancies[0] * active[0].astype(dtype)).astype(dtype)
  dens = radial_densities[0].astype(dtype) * occ[:, None]

  g = grid_to_cartesian.astype(dtype)
  rstep = jnp.asarray(RSTEP, dtype)
  hwx = (RMAX / jnp.abs(g[0, 0])) * 1.00002 + 1e-3
  nchx = jnp.ceil((2.0 * hwx + 2.0) / L).astype(jnp.int32)
  clampx = jnp.maximum(0.0, jnp.asarray(GRID, dtype) - L * nchx.astype(dtype))
  gv = jnp.zeros((16,), dtype) \
      .at[0].set(g[0, 0]).at[1].set(g[0, 1]).at[2].set(g[0, 2]) \
      .at[3].set(g[1, 1]).at[4].set(g[1, 2]).at[5].set(g[2, 2]) \
      .at[6].set(-g[1, 2] / g[1, 1]) \
      .at[7].set(1.0 / jnp.abs(g[1, 1])) \
      .at[8].set(1.0 / rstep) \
      .at[9].set(-1.0 / g[0, 0]) \
      .at[10].set(hwx) \
      .at[11].set(clampx) \
      .at[12].set(nchx.astype(dtype))

  mesh = plsc.VectorSubcoreMesh(core_axis_name="c", subcore_axis_name="s")
  run = pl.kernel(
      _sc_body,
      out_type=jax.ShapeDtypeStruct((GRID, PLANE), dtype),
      mesh=mesh,
      compiler_params=pltpu.CompilerParams(needs_layout_passes=False),
      scratch_types=[
          pltpu.VMEM((NATOMS,), dtype),
          pltpu.VMEM((NATOMS,), dtype),
          pltpu.VMEM((NATOMS,), dtype),
          pltpu.VMEM((NATOMS, NRAD), dtype),
          pltpu.VMEM((L,), dtype),
          pltpu.VMEM((PLANE + L,), dtype),
          pltpu.SemaphoreType.DMA,
      ],
  )
  out = run(ax, ay, az, dens, gv)
  return out.reshape((1, GRID, GRID, GRID))


# speedup vs baseline: 1.0692x; 1.0692x over previous
import jax
import jax.numpy as jnp
from jax import lax
from jax.experimental import pallas as pl
from jax.experimental.pallas import tpu as pltpu
from jax.experimental.pallas import tpu_sc as plsc

GRID = 32
RSTEP = 0.1
RMAX = 3.0
NATOMS = 128
NRAD = 64
L = 16


def _splat(vec, j):
  return jnp.full((L,), vec[j], dtype=vec.dtype)


def _sc_body(ax_h, ay_h, az_h, dens_h, gv_h, out_h,
             ax_v, ay_v, az_v, dens_v, gv_v, plane_v, sem):
  cid = lax.axis_index("c")
  sid = lax.axis_index("s")
  wid = sid * 2 + cid

  pltpu.async_copy(dens_h, dens_v, sem).wait()
  pltpu.async_copy(ax_h, ax_v, sem).wait()
  pltpu.async_copy(ay_h, ay_v, sem).wait()
  pltpu.async_copy(az_h, az_v, sem).wait()
  pltpu.async_copy(gv_h, gv_v, sem).wait()

  gv = gv_v[...]
  g00 = _splat(gv, 0)
  g01 = _splat(gv, 1)
  g02 = _splat(gv, 2)
  g11 = _splat(gv, 3)
  g12 = _splat(gv, 4)
  g22 = _splat(gv, 5)
  ngd = _splat(gv, 6)
  invg11 = _splat(gv, 7)
  inv_rstep = _splat(gv, 8)

  iota = lax.iota(jnp.int32, L)
  xf0 = iota.astype(jnp.float32)
  xf1 = (iota + 16).astype(jnp.float32)
  g00x0 = g00 * xf0
  g00x1 = g00 * xf1

  zero16 = jnp.zeros((L,), jnp.float32)

  def zero_body(r, _):
    plane_v[r] = zero16
    return _

  lax.fori_loop(0, 2 * GRID, zero_body, None)

  rmax2 = jnp.full((L,), RMAX * RMAX, jnp.float32)
  rmax2_pad = jnp.full((L,), RMAX * RMAX + 1e-3, jnp.float32)
  half = jnp.full((L,), 0.5, jnp.float32)
  three_half = jnp.full((L,), 1.5, jnp.float32)
  magic = jnp.full((L,), 0x5F3759DF, jnp.int32)
  one_i = jnp.full((L,), 1, jnp.int32)
  zero_i = jnp.full((L,), 0, jnp.int32)

  def newton_rsqrt(a):
    bits = plsc.bitcast(a, jnp.int32)
    y0 = plsc.bitcast(magic - lax.shift_right_logical(bits, 1), jnp.float32)
    hx = half * a
    y0 = y0 * (three_half - hx * y0 * y0)
    y0 = y0 * (three_half - hx * y0 * y0)
    return y0

  def half_pass(zplane, ymin, ymax):
    zfs = jnp.full((L,), zplane, dtype=jnp.int32).astype(jnp.float32)
    ymin_f = jnp.full((L,), ymin, jnp.int32).astype(jnp.float32)
    ymax_f = jnp.full((L,), ymax, jnp.int32).astype(jnp.float32)

    def chunk_body(c, _):
      base = c * L
      axv = ax_v[pl.ds(base, L)]
      ayv = ay_v[pl.ds(base, L)]
      azv = az_v[pl.ds(base, L)]

      dzv = zfs - azv
      cdzv = g22 * dzv
      remy = rmax2_pad - cdzv * cdzv
      ok = remy >= 0.0
      remy_nn = jnp.maximum(remy, 0.0)
      sy = remy_nn * newton_rsqrt(remy_nn)
      sy = sy * 1.00002 + 1e-3
      sy = jnp.where(ok, sy, -1.0)
      cyv = ngd * dzv
      hw = sy * invg11
      ylo_f = jnp.maximum(ayv + cyv - hw, ymin_f)
      yhi_f = jnp.minimum(ayv + cyv + hw, ymax_f)
      ilo = ylo_f.astype(jnp.int32)
      ylov = ilo + jnp.where(ilo.astype(jnp.float32) < ylo_f, one_i, zero_i)
      ycntv = yhi_f.astype(jnp.int32) - ylov + 1

      for j in range(L):
        ycnt_s = ycntv[j]

        @pl.when(ycnt_s > 0)
        def _():
          n = base + j
          nv = jnp.full((L,), n, jnp.int32)
          axs = _splat(axv, j)
          ays = _splat(ayv, j)
          dzs = _splat(dzv, j)
          cdzs = _splat(cdzv, j)
          ylo_s = ylov[j]
          cdz2 = cdzs * cdzs
          g12dz = g12 * dzs
          g02dz = g02 * dzs
          g00ax = g00 * axs

          def row_body(yi, _c):
            y = ylo_s + yi
            dyv = jnp.full((L,), y, jnp.int32).astype(jnp.float32) - ays
            cdy = g12dz + g11 * dyv
            cyz2 = cdz2 + cdy * cdy
            rowbase = (g02dz + g01 * dyv) - g00ax
            r = y * 2

            def do_half(hh, g00xf):
              cdx = rowbase + g00xf
              d2 = cdx * cdx + cyz2
              m = d2 <= rmax2
              y0 = newton_rsqrt(d2)
              dist = d2 * y0
              rad = dist * inv_rstep
              il_raw = rad.astype(jnp.int32)
              wh = rad - il_raw.astype(jnp.float32)
              il = jnp.minimum(il_raw, NRAD - 1)
              ih = jnp.minimum(il_raw + 1, NRAD - 1)
              dl = plsc.load_gather(dens_v, [nv, il])
              dh = plsc.load_gather(dens_v, [nv, ih])
              dens = dl + wh * (dh - dl)
              contrib = jnp.where(m, dens, 0.0)
              plsc.addupdate(plane_v.at[r + hh], contrib)

            do_half(0, g00x0)
            do_half(1, g00x1)
            return _c

          lax.fori_loop(0, ycnt_s, row_body, None)

      return _

    lax.fori_loop(0, NATOMS // L, chunk_body, None)

  z1 = wid
  z2 = (GRID - 1) - wid
  half_pass(z1, 0, GRID // 2 - 1)
  half_pass(z2, GRID // 2, GRID - 1)

  pltpu.async_copy(plane_v.at[pl.ds(0, GRID)],
                   out_h.at[z1, pl.ds(0, GRID)], sem).wait()
  pltpu.async_copy(plane_v.at[pl.ds(GRID, GRID)],
                   out_h.at[z2, pl.ds(GRID, GRID)], sem).wait()


def kernel(coordinates, active, occupancies, lmax, radial_densities,
           grid_to_cartesian):
  del lmax
  dtype = jnp.float32
  coords = coordinates[0].astype(dtype)
  ax = coords[:, 0]
  ay = coords[:, 1]
  az = coords[:, 2]
  occ = (occupancies[0] * active[0].astype(dtype)).astype(dtype)
  dens = radial_densities[0].astype(dtype) * occ[:, None]

  g = grid_to_cartesian.astype(dtype)
  rstep = jnp.asarray(RSTEP, dtype)
  gv = jnp.zeros((16,), dtype) \
      .at[0].set(g[0, 0]).at[1].set(g[0, 1]).at[2].set(g[0, 2]) \
      .at[3].set(g[1, 1]).at[4].set(g[1, 2]).at[5].set(g[2, 2]) \
      .at[6].set(-g[1, 2] / g[1, 1]) \
      .at[7].set(1.0 / jnp.abs(g[1, 1])) \
      .at[8].set(1.0 / rstep)

  mesh = plsc.VectorSubcoreMesh(core_axis_name="c", subcore_axis_name="s")
  run = pl.kernel(
      _sc_body,
      out_type=jax.ShapeDtypeStruct((GRID, 2 * GRID, L), dtype),
      mesh=mesh,
      compiler_params=pltpu.CompilerParams(needs_layout_passes=False),
      scratch_types=[
          pltpu.VMEM((NATOMS,), dtype),
          pltpu.VMEM((NATOMS,), dtype),
          pltpu.VMEM((NATOMS,), dtype),
          pltpu.VMEM((NATOMS, NRAD), dtype),
          pltpu.VMEM((L,), dtype),
          pltpu.VMEM((2 * GRID, L), dtype),
          pltpu.SemaphoreType.DMA,
      ],
  )
  out = run(ax, ay, az, dens, gv)
  return out.reshape((1, GRID, GRID, GRID))


# speedup vs baseline: 1.1205x; 1.0480x over previous
import jax
import jax.numpy as jnp
from jax import lax
from jax.experimental import pallas as pl
from jax.experimental.pallas import tpu as pltpu
from jax.experimental.pallas import tpu_sc as plsc

GRID = 32
RSTEP = 0.1
RMAX = 3.0
NATOMS = 128
NRAD = 64
L = 16


def _splat(vec, j):
  return jnp.full((L,), vec[j], dtype=vec.dtype)


def _sc_body(ax_h, ay_h, az_h, dens_h, gv_h, out_h,
             ax_v, ay_v, az_v, dens_v, gv_v, plane_v, sem):
  cid = lax.axis_index("c")
  sid = lax.axis_index("s")
  wid = sid * 2 + cid

  pltpu.async_copy(dens_h, dens_v, sem).wait()
  pltpu.async_copy(ax_h, ax_v, sem).wait()
  pltpu.async_copy(ay_h, ay_v, sem).wait()
  pltpu.async_copy(az_h, az_v, sem).wait()
  pltpu.async_copy(gv_h, gv_v, sem).wait()

  gv = gv_v[...]
  g00 = _splat(gv, 0)
  g01 = _splat(gv, 1)
  g02 = _splat(gv, 2)
  g11 = _splat(gv, 3)
  g12 = _splat(gv, 4)
  g22 = _splat(gv, 5)
  ngd = _splat(gv, 6)
  invg11 = _splat(gv, 7)
  inv_rstep = _splat(gv, 8)

  iota = lax.iota(jnp.int32, L)
  xf0 = iota.astype(jnp.float32)
  xf1 = (iota + 16).astype(jnp.float32)
  g00x0 = g00 * xf0
  g00x1 = g00 * xf1

  zero16 = jnp.zeros((L,), jnp.float32)

  def zero_body(r, _):
    plane_v[r] = zero16
    return _

  lax.fori_loop(0, 2 * GRID, zero_body, None)

  rmax2 = jnp.full((L,), RMAX * RMAX, jnp.float32)
  rmax2_pad = jnp.full((L,), RMAX * RMAX + 1e-3, jnp.float32)
  half = jnp.full((L,), 0.5, jnp.float32)
  three_half = jnp.full((L,), 1.5, jnp.float32)
  magic = jnp.full((L,), 0x5F3759DF, jnp.int32)
  one_i = jnp.full((L,), 1, jnp.int32)
  zero_i = jnp.full((L,), 0, jnp.int32)

  def newton_rsqrt(a):
    bits = plsc.bitcast(a, jnp.int32)
    y0 = plsc.bitcast(magic - lax.shift_right_logical(bits, 1), jnp.float32)
    hx = half * a
    y0 = y0 * (three_half - hx * y0 * y0)
    y0 = y0 * (three_half - hx * y0 * y0)
    return y0

  z1 = wid
  z2 = (GRID - 1) - wid

  def half_pass(p, _p):
    zplane = jnp.where(p == 0, z1, z2)
    ymin = p * (GRID // 2)
    ymax = ymin + GRID // 2 - 1
    zfs = jnp.full((L,), zplane, dtype=jnp.int32).astype(jnp.float32)
    ymin_f = jnp.full((L,), ymin, jnp.int32).astype(jnp.float32)
    ymax_f = jnp.full((L,), ymax, jnp.int32).astype(jnp.float32)

    def chunk_body(c, _):
      base = c * L
      axv = ax_v[pl.ds(base, L)]
      ayv = ay_v[pl.ds(base, L)]
      azv = az_v[pl.ds(base, L)]

      dzv = zfs - azv
      cdzv = g22 * dzv
      remy = rmax2_pad - cdzv * cdzv
      ok = remy >= 0.0
      remy_nn = jnp.maximum(remy, 0.0)
      sy = remy_nn * newton_rsqrt(remy_nn)
      sy = sy * 1.00002 + 1e-3
      sy = jnp.where(ok, sy, -1.0)
      cyv = ngd * dzv
      hw = sy * invg11
      ylo_f = jnp.maximum(ayv + cyv - hw, ymin_f)
      yhi_f = jnp.minimum(ayv + cyv + hw, ymax_f)
      ilo = ylo_f.astype(jnp.int32)
      ylov = ilo + jnp.where(ilo.astype(jnp.float32) < ylo_f, one_i, zero_i)
      ycntv = yhi_f.astype(jnp.int32) - ylov + 1

      for j in range(L):
        ycnt_s = ycntv[j]

        @pl.when(ycnt_s > 0)
        def _():
          n = base + j
          nv = jnp.full((L,), n, jnp.int32)
          axs = _splat(axv, j)
          ays = _splat(ayv, j)
          dzs = _splat(dzv, j)
          cdzs = _splat(cdzv, j)
          ylo_s = ylov[j]
          cdz2 = cdzs * cdzs
          g12dz = g12 * dzs
          g02dz = g02 * dzs
          g00ax = g00 * axs

          def row_body(yi, _c):
            y = ylo_s + yi
            dyv = jnp.full((L,), y, jnp.int32).astype(jnp.float32) - ays
            cdy = g12dz + g11 * dyv
            cyz2 = cdz2 + cdy * cdy
            rowbase = (g02dz + g01 * dyv) - g00ax
            r = y * 2

            def do_half(hh, g00xf):
              cdx = rowbase + g00xf
              d2 = cdx * cdx + cyz2
              m = d2 <= rmax2
              y0 = newton_rsqrt(d2)
              dist = d2 * y0
              rad = dist * inv_rstep
              il_raw = rad.astype(jnp.int32)
              wh = rad - il_raw.astype(jnp.float32)
              il = jnp.minimum(il_raw, NRAD - 1)
              ih = jnp.minimum(il_raw + 1, NRAD - 1)
              dl = plsc.load_gather(dens_v, [nv, il])
              dh = plsc.load_gather(dens_v, [nv, ih])
              dens = dl + wh * (dh - dl)
              contrib = jnp.where(m, dens, 0.0)
              plsc.addupdate(plane_v.at[r + hh], contrib)

            do_half(0, g00x0)
            do_half(1, g00x1)
            return _c

          lax.fori_loop(0, ycnt_s, row_body, None)

      return _

    lax.fori_loop(0, NATOMS // L, chunk_body, None)
    return _p

  lax.fori_loop(0, 2, half_pass, None)

  pltpu.async_copy(plane_v.at[pl.ds(0, GRID)],
                   out_h.at[z1, pl.ds(0, GRID)], sem).wait()
  pltpu.async_copy(plane_v.at[pl.ds(GRID, GRID)],
                   out_h.at[z2, pl.ds(GRID, GRID)], sem).wait()


def kernel(coordinates, active, occupancies, lmax, radial_densities,
           grid_to_cartesian):
  del lmax
  dtype = jnp.float32
  coords = coordinates[0].astype(dtype)
  ax = coords[:, 0]
  ay = coords[:, 1]
  az = coords[:, 2]
  occ = (occupancies[0] * active[0].astype(dtype)).astype(dtype)
  dens = radial_densities[0].astype(dtype) * occ[:, None]

  g = grid_to_cartesian.astype(dtype)
  rstep = jnp.asarray(RSTEP, dtype)
  gv = jnp.zeros((16,), dtype) \
      .at[0].set(g[0, 0]).at[1].set(g[0, 1]).at[2].set(g[0, 2]) \
      .at[3].set(g[1, 1]).at[4].set(g[1, 2]).at[5].set(g[2, 2]) \
      .at[6].set(-g[1, 2] / g[1, 1]) \
      .at[7].set(1.0 / jnp.abs(g[1, 1])) \
      .at[8].set(1.0 / rstep)

  mesh = plsc.VectorSubcoreMesh(core_axis_name="c", subcore_axis_name="s")
  run = pl.kernel(
      _sc_body,
      out_type=jax.ShapeDtypeStruct((GRID, 2 * GRID, L), dtype),
      mesh=mesh,
      compiler_params=pltpu.CompilerParams(needs_layout_passes=False),
      scratch_types=[
          pltpu.VMEM((NATOMS,), dtype),
          pltpu.VMEM((NATOMS,), dtype),
          pltpu.VMEM((NATOMS,), dtype),
          pltpu.VMEM((NATOMS, NRAD), dtype),
          pltpu.VMEM((L,), dtype),
          pltpu.VMEM((2 * GRID, L), dtype),
          pltpu.SemaphoreType.DMA,
      ],
  )
  out = run(ax, ay, az, dens, gv)
  return out.reshape((1, GRID, GRID, GRID))


# speedup vs baseline: 1.1604x; 1.0356x over previous
import jax
import jax.numpy as jnp
from jax import lax
from jax.experimental import pallas as pl
from jax.experimental.pallas import tpu as pltpu
from jax.experimental.pallas import tpu_sc as plsc

GRID = 32
RSTEP = 0.1
RMAX = 3.0
NATOMS = 128
NRAD = 64
L = 16


def _splat(vec, j):
  return jnp.full((L,), vec[j], dtype=vec.dtype)


def _sc_body(ax_h, ay_h, az_h, dens_h, gv_h, out_h,
             ax_v, ay_v, az_v, dens_v, gv_v, plane_v, sem):
  cid = lax.axis_index("c")
  sid = lax.axis_index("s")
  wid = sid * 2 + cid

  pltpu.async_copy(dens_h, dens_v, sem).wait()
  pltpu.async_copy(ax_h, ax_v, sem).wait()
  pltpu.async_copy(ay_h, ay_v, sem).wait()
  pltpu.async_copy(az_h, az_v, sem).wait()
  pltpu.async_copy(gv_h, gv_v, sem).wait()

  gv = gv_v[...]
  g00 = _splat(gv, 0)
  g01 = _splat(gv, 1)
  g02 = _splat(gv, 2)
  g11 = _splat(gv, 3)
  g12 = _splat(gv, 4)
  g22 = _splat(gv, 5)
  ngd = _splat(gv, 6)
  invg11 = _splat(gv, 7)
  inv_rstep = _splat(gv, 8)

  zf = jnp.full((L,), wid, dtype=jnp.int32).astype(jnp.float32)
  iota = lax.iota(jnp.int32, L)
  xf0 = iota.astype(jnp.float32)
  xf1 = (iota + 16).astype(jnp.float32)
  g00x0 = g00 * xf0
  g00x1 = g00 * xf1

  zero16 = jnp.zeros((L,), jnp.float32)

  def zero_body(r, _):
    plane_v[r] = zero16
    return _

  lax.fori_loop(0, 2 * GRID + 4, zero_body, None)

  rmax2 = jnp.full((L,), RMAX * RMAX, jnp.float32)
  rmax2_pad = jnp.full((L,), RMAX * RMAX + 1e-3, jnp.float32)
  half = jnp.full((L,), 0.5, jnp.float32)
  three_half = jnp.full((L,), 1.5, jnp.float32)
  magic = jnp.full((L,), 0x5F3759DF, jnp.int32)
  one_i = jnp.full((L,), 1, jnp.int32)
  zero_i = jnp.full((L,), 0, jnp.int32)

  def newton_rsqrt(a):
    bits = plsc.bitcast(a, jnp.int32)
    y0 = plsc.bitcast(magic - lax.shift_right_logical(bits, 1), jnp.float32)
    hx = half * a
    y0 = y0 * (three_half - hx * y0 * y0)
    y0 = y0 * (three_half - hx * y0 * y0)
    return y0

  def chunk_body(c, _):
    base = c * L
    axv = ax_v[pl.ds(base, L)]
    ayv = ay_v[pl.ds(base, L)]
    azv = az_v[pl.ds(base, L)]

    dzv = zf - azv
    cdzv = g22 * dzv
    remy = rmax2_pad - cdzv * cdzv
    ok = remy >= 0.0
    remy_nn = jnp.maximum(remy, 0.0)
    sy = remy_nn * newton_rsqrt(remy_nn)
    sy = sy * 1.00002 + 1e-3
    sy = jnp.where(ok, sy, -1.0)
    cyv = ngd * dzv
    hw = sy * invg11
    ylo_f = jnp.maximum(ayv + cyv - hw, 0.0)
    yhi_f = jnp.minimum(ayv + cyv + hw, GRID - 1.0)
    ilo = ylo_f.astype(jnp.int32)
    ylov = ilo + jnp.where(ilo.astype(jnp.float32) < ylo_f, one_i, zero_i)
    ycntv = yhi_f.astype(jnp.int32) - ylov + 1

    for j in range(L):
      ycnt_s = ycntv[j]

      @pl.when(ycnt_s > 0)
      def _():
        n = base + j
        nv = jnp.full((L,), n, jnp.int32)
        axs = _splat(axv, j)
        ays = _splat(ayv, j)
        dzs = _splat(dzv, j)
        cdzs = _splat(cdzv, j)
        ylo_s = ylov[j]
        cdz2 = cdzs * cdzs
        g12dz = g12 * dzs
        g02dz = g02 * dzs
        g00ax = g00 * axs

        @plsc.parallel_loop(0, ycnt_s, unroll=2)
        def row_body(yi):
          y = ylo_s + yi

          def one_row(yy):
            dyv = jnp.full((L,), yy, jnp.int32).astype(jnp.float32) - ays
            cdy = g12dz + g11 * dyv
            cyz2 = cdz2 + cdy * cdy
            rowbase = (g02dz + g01 * dyv) - g00ax
            r = yy * 2

            def do_half(hh, g00xf):
              cdx = rowbase + g00xf
              d2 = cdx * cdx + cyz2
              m = d2 <= rmax2
              y0 = newton_rsqrt(d2)
              dist = d2 * y0
              rad = dist * inv_rstep
              il_raw = rad.astype(jnp.int32)
              wh = rad - il_raw.astype(jnp.float32)
              il = jnp.minimum(il_raw, NRAD - 1)
              ih = jnp.minimum(il_raw + 1, NRAD - 1)
              dl = plsc.load_gather(dens_v, [nv, il])
              dh = plsc.load_gather(dens_v, [nv, ih])
              dens = dl + wh * (dh - dl)
              contrib = jnp.where(m, dens, 0.0)
              plsc.addupdate(plane_v.at[r + hh], contrib)

            do_half(0, g00x0)
            do_half(1, g00x1)

          one_row(y)

    return _

  lax.fori_loop(0, NATOMS // L, chunk_body, None)

  pltpu.async_copy(plane_v.at[pl.ds(0, 2 * GRID)],
                   out_h.at[wid], sem).wait()


def kernel(coordinates, active, occupancies, lmax, radial_densities,
           grid_to_cartesian):
  del lmax
  dtype = jnp.float32
  coords = coordinates[0].astype(dtype)
  ax = coords[:, 0]
  ay = coords[:, 1]
  az = coords[:, 2]
  occ = (occupancies[0] * active[0].astype(dtype)).astype(dtype)
  dens = radial_densities[0].astype(dtype) * occ[:, None]

  g = grid_to_cartesian.astype(dtype)
  rstep = jnp.asarray(RSTEP, dtype)
  gv = jnp.zeros((16,), dtype) \
      .at[0].set(g[0, 0]).at[1].set(g[0, 1]).at[2].set(g[0, 2]) \
      .at[3].set(g[1, 1]).at[4].set(g[1, 2]).at[5].set(g[2, 2]) \
      .at[6].set(-g[1, 2] / g[1, 1]) \
      .at[7].set(1.0 / jnp.abs(g[1, 1])) \
      .at[8].set(1.0 / rstep)

  mesh = plsc.VectorSubcoreMesh(core_axis_name="c", subcore_axis_name="s")
  run = pl.kernel(
      _sc_body,
      out_type=jax.ShapeDtypeStruct((GRID, 2 * GRID, L), dtype),
      mesh=mesh,
      compiler_params=pltpu.CompilerParams(needs_layout_passes=False),
      scratch_types=[
          pltpu.VMEM((NATOMS,), dtype),
          pltpu.VMEM((NATOMS,), dtype),
          pltpu.VMEM((NATOMS,), dtype),
          pltpu.VMEM((NATOMS, NRAD), dtype),
          pltpu.VMEM((L,), dtype),
          pltpu.VMEM((2 * GRID + 4, L), dtype),
          pltpu.SemaphoreType.DMA,
      ],
  )
  out = run(ax, ay, az, dens, gv)
  return out.reshape((1, GRID, GRID, GRID))


# speedup vs baseline: 1.4322x; 1.2342x over previous
import jax
import jax.numpy as jnp
from jax import lax
from jax.experimental import pallas as pl
from jax.experimental.pallas import tpu as pltpu
from jax.experimental.pallas import tpu_sc as plsc

GRID = 32
RSTEP = 0.1
RMAX = 3.0
NATOMS = 128
NRAD = 64
L = 16


def _splat(vec, j):
  return jnp.full((L,), vec[j], dtype=vec.dtype)


def _sc_body(ax_h, ay_h, az_h, dens_h, gv_h, out_h,
             ax_v, ay_v, az_v, dens_v, gv_v, plane_v, sem):
  cid = lax.axis_index("c")
  sid = lax.axis_index("s")
  wid = sid * 2 + cid

  pltpu.async_copy(dens_h, dens_v, sem).wait()
  pltpu.async_copy(ax_h, ax_v, sem).wait()
  pltpu.async_copy(ay_h, ay_v, sem).wait()
  pltpu.async_copy(az_h, az_v, sem).wait()
  pltpu.async_copy(gv_h, gv_v, sem).wait()

  gv = gv_v[...]
  g00 = _splat(gv, 0)
  g01 = _splat(gv, 1)
  g02 = _splat(gv, 2)
  g11 = _splat(gv, 3)
  g12 = _splat(gv, 4)
  g22 = _splat(gv, 5)
  ngd = _splat(gv, 6)
  invg11 = _splat(gv, 7)
  inv_rstep = _splat(gv, 8)

  zf = jnp.full((L,), wid, dtype=jnp.int32).astype(jnp.float32)
  iota = lax.iota(jnp.int32, L)
  xf0 = iota.astype(jnp.float32)
  xf1 = (iota + 16).astype(jnp.float32)
  g00x0 = g00 * xf0
  g00x1 = g00 * xf1

  zero16 = jnp.zeros((L,), jnp.float32)

  def zero_body(r, _):
    plane_v[r] = zero16
    return _

  lax.fori_loop(0, 2 * GRID + 4, zero_body, None)

  rmax2 = jnp.full((L,), RMAX * RMAX, jnp.float32)
  rmax2_pad = jnp.full((L,), RMAX * RMAX + 1e-3, jnp.float32)
  half = jnp.full((L,), 0.5, jnp.float32)
  three_half = jnp.full((L,), 1.5, jnp.float32)
  magic = jnp.full((L,), 0x5F3759DF, jnp.int32)
  one_i = jnp.full((L,), 1, jnp.int32)
  zero_i = jnp.full((L,), 0, jnp.int32)

  def newton_rsqrt(a):
    bits = plsc.bitcast(a, jnp.int32)
    y0 = plsc.bitcast(magic - lax.shift_right_logical(bits, 1), jnp.float32)
    hx = half * a
    y0 = y0 * (three_half - hx * y0 * y0)
    y0 = y0 * (three_half - hx * y0 * y0)
    return y0

  def chunk_body(c, _):
    base = c * L
    axv = ax_v[pl.ds(base, L)]
    ayv = ay_v[pl.ds(base, L)]
    azv = az_v[pl.ds(base, L)]

    dzv = zf - azv
    cdzv = g22 * dzv
    remy = rmax2_pad - cdzv * cdzv
    ok = remy >= 0.0
    remy_nn = jnp.maximum(remy, 0.0)
    sy = remy_nn * newton_rsqrt(remy_nn)
    sy = sy * 1.00002 + 1e-3
    sy = jnp.where(ok, sy, -1.0)
    cyv = ngd * dzv
    hw = sy * invg11
    ylo_f = jnp.maximum(ayv + cyv - hw, 0.0)
    yhi_f = jnp.minimum(ayv + cyv + hw, GRID - 1.0)
    ilo = ylo_f.astype(jnp.int32)
    ylov = ilo + jnp.where(ilo.astype(jnp.float32) < ylo_f, one_i, zero_i)
    ycntv = yhi_f.astype(jnp.int32) - ylov + 1

    for j in range(L):
      ycnt_s = ycntv[j]

      @pl.when(ycnt_s > 0)
      def _():
        n = base + j
        nv = jnp.full((L,), n, jnp.int32)
        axs = _splat(axv, j)
        ays = _splat(ayv, j)
        dzs = _splat(dzv, j)
        cdzs = _splat(cdzv, j)
        ylo_s = ylov[j]
        cdz2 = cdzs * cdzs
        g12dz = g12 * dzs
        g02dz = g02 * dzs
        g00ax = g00 * axs

        def row_body(yi, _c):
          y = ylo_s + yi

          def one_row(yy):
            dyv = jnp.full((L,), yy, jnp.int32).astype(jnp.float32) - ays
            cdy = g12dz + g11 * dyv
            cyz2 = cdz2 + cdy * cdy
            rowbase = (g02dz + g01 * dyv) - g00ax
            r = yy * 2

            def do_half(hh, g00xf):
              cdx = rowbase + g00xf
              d2 = cdx * cdx + cyz2
              m = d2 <= rmax2
              y0 = newton_rsqrt(d2)
              dist = d2 * y0
              rad = dist * inv_rstep
              il_raw = rad.astype(jnp.int32)
              wh = rad - il_raw.astype(jnp.float32)
              il = jnp.minimum(il_raw, NRAD - 1)
              ih = jnp.minimum(il_raw + 1, NRAD - 1)
              dl = plsc.load_gather(dens_v, [nv, il])
              dh = plsc.load_gather(dens_v, [nv, ih])
              dens = dl + wh * (dh - dl)
              contrib = jnp.where(m, dens, 0.0)
              plsc.addupdate(plane_v.at[r + hh], contrib)

            do_half(0, g00x0)
            do_half(1, g00x1)

          one_row(y)
          return _c

        lax.fori_loop(0, ycnt_s, row_body, None)

    return _

  lax.fori_loop(0, NATOMS // L, chunk_body, None)

  pltpu.async_copy(plane_v.at[pl.ds(0, 2 * GRID)],
                   out_h.at[wid], sem).wait()


def kernel(coordinates, active, occupancies, lmax, radial_densities,
           grid_to_cartesian):
  del lmax
  dtype = jnp.float32
  coords = coordinates[0].astype(dtype)
  ax = coords[:, 0]
  ay = coords[:, 1]
  az = coords[:, 2]
  occ = (occupancies[0] * active[0].astype(dtype)).astype(dtype)
  dens = radial_densities[0].astype(dtype) * occ[:, None]

  g = grid_to_cartesian.astype(dtype)
  rstep = jnp.asarray(RSTEP, dtype)
  gv = jnp.zeros((16,), dtype) \
      .at[0].set(g[0, 0]).at[1].set(g[0, 1]).at[2].set(g[0, 2]) \
      .at[3].set(g[1, 1]).at[4].set(g[1, 2]).at[5].set(g[2, 2]) \
      .at[6].set(-g[1, 2] / g[1, 1]) \
      .at[7].set(1.0 / jnp.abs(g[1, 1])) \
      .at[8].set(1.0 / rstep)

  mesh = plsc.VectorSubcoreMesh(core_axis_name="c", subcore_axis_name="s")
  run = pl.kernel(
      _sc_body,
      out_type=jax.ShapeDtypeStruct((GRID, 2 * GRID, L), dtype),
      mesh=mesh,
      compiler_params=pltpu.CompilerParams(needs_layout_passes=False),
      scratch_types=[
          pltpu.VMEM((NATOMS,), dtype),
          pltpu.VMEM((NATOMS,), dtype),
          pltpu.VMEM((NATOMS,), dtype),
          pltpu.VMEM((NATOMS, NRAD), dtype),
          pltpu.VMEM((L,), dtype),
          pltpu.VMEM((2 * GRID + 4, L), dtype),
          pltpu.SemaphoreType.DMA,
      ],
  )
  out = run(ax, ay, az, dens, gv)
  return out.reshape((1, GRID, GRID, GRID))


# speedup vs baseline: 1.4641x; 1.0223x over previous
import jax
import jax.numpy as jnp
from jax import lax
from jax.experimental import pallas as pl
from jax.experimental.pallas import tpu as pltpu
from jax.experimental.pallas import tpu_sc as plsc

GRID = 32
RSTEP = 0.1
RMAX = 3.0
NATOMS = 128
NRAD = 64
L = 16


def _splat(vec, j):
  return jnp.full((L,), vec[j], dtype=vec.dtype)


def _sc_body(pack_h, dens_h, out_h, pack_v, dens_v, plane_v, sem):
  cid = lax.axis_index("c")
  sid = lax.axis_index("s")
  wid = sid * 2 + cid

  c1 = pltpu.async_copy(dens_h, dens_v, sem)
  c2 = pltpu.async_copy(pack_h, pack_v, sem)
  c1.wait()
  c2.wait()

  gv = pack_v[pl.ds(3 * NATOMS, L)]
  g00 = _splat(gv, 0)
  g01 = _splat(gv, 1)
  g02 = _splat(gv, 2)
  g11 = _splat(gv, 3)
  g12 = _splat(gv, 4)
  g22 = _splat(gv, 5)
  ngd = _splat(gv, 6)
  invg11 = _splat(gv, 7)
  inv_rstep = _splat(gv, 8)

  zf = jnp.full((L,), wid, dtype=jnp.int32).astype(jnp.float32)
  iota = lax.iota(jnp.int32, L)
  xf0 = iota.astype(jnp.float32)
  xf1 = (iota + 16).astype(jnp.float32)
  g00x0 = g00 * xf0
  g00x1 = g00 * xf1

  zero16 = jnp.zeros((L,), jnp.float32)

  def zero_body(r, _):
    plane_v[r] = zero16
    return _

  lax.fori_loop(0, 2 * GRID + 4, zero_body, None)

  rmax2 = jnp.full((L,), RMAX * RMAX, jnp.float32)
  rmax2_pad = jnp.full((L,), RMAX * RMAX + 1e-3, jnp.float32)
  half = jnp.full((L,), 0.5, jnp.float32)
  three_half = jnp.full((L,), 1.5, jnp.float32)
  magic = jnp.full((L,), 0x5F3759DF, jnp.int32)
  one_i = jnp.full((L,), 1, jnp.int32)
  zero_i = jnp.full((L,), 0, jnp.int32)

  def newton_rsqrt(a):
    bits = plsc.bitcast(a, jnp.int32)
    y0 = plsc.bitcast(magic - lax.shift_right_logical(bits, 1), jnp.float32)
    hx = half * a
    y0 = y0 * (three_half - hx * y0 * y0)
    y0 = y0 * (three_half - hx * y0 * y0)
    return y0

  def chunk_body(c, _):
    base = c * L
    axv = pack_v[pl.ds(base, L)]
    ayv = pack_v[pl.ds(NATOMS + base, L)]
    azv = pack_v[pl.ds(2 * NATOMS + base, L)]

    dzv = zf - azv
    cdzv = g22 * dzv
    remy = rmax2_pad - cdzv * cdzv
    ok = remy >= 0.0
    remy_nn = jnp.maximum(remy, 0.0)
    sy = remy_nn * newton_rsqrt(remy_nn)
    sy = sy * 1.00002 + 1e-3
    sy = jnp.where(ok, sy, -1.0)
    cyv = ngd * dzv
    hw = sy * invg11
    ylo_f = jnp.maximum(ayv + cyv - hw, 0.0)
    yhi_f = jnp.minimum(ayv + cyv + hw, GRID - 1.0)
    ilo = ylo_f.astype(jnp.int32)
    ylov = ilo + jnp.where(ilo.astype(jnp.float32) < ylo_f, one_i, zero_i)
    ycntv = yhi_f.astype(jnp.int32) - ylov + 1

    for j in range(L):
      ycnt_s = ycntv[j]

      @pl.when(ycnt_s > 0)
      def _():
        n = base + j
        nv = jnp.full((L,), n, jnp.int32)
        axs = _splat(axv, j)
        ays = _splat(ayv, j)
        dzs = _splat(dzv, j)
        cdzs = _splat(cdzv, j)
        ylo_s = ylov[j]
        cdz2 = cdzs * cdzs
        g12dz = g12 * dzs
        g02dz = g02 * dzs
        g00ax = g00 * axs

        def row_body(yi, _c):
          y = ylo_s + yi

          def one_row(yy):
            dyv = jnp.full((L,), yy, jnp.int32).astype(jnp.float32) - ays
            cdy = g12dz + g11 * dyv
            cyz2 = cdz2 + cdy * cdy
            rowbase = (g02dz + g01 * dyv) - g00ax
            r = yy * 2

            def do_half(hh, g00xf):
              cdx = rowbase + g00xf
              d2 = cdx * cdx + cyz2
              m = d2 <= rmax2
              y0 = newton_rsqrt(d2)
              dist = d2 * y0
              rad = dist * inv_rstep
              il_raw = rad.astype(jnp.int32)
              wh = rad - il_raw.astype(jnp.float32)
              il = jnp.minimum(il_raw, NRAD - 1)
              ih = jnp.minimum(il_raw + 1, NRAD - 1)
              dl = plsc.load_gather(dens_v, [nv, il])
              dh = plsc.load_gather(dens_v, [nv, ih])
              dens = dl + wh * (dh - dl)
              contrib = jnp.where(m, dens, 0.0)
              plsc.addupdate(plane_v.at[r + hh], contrib)

            do_half(0, g00x0)
            do_half(1, g00x1)

          one_row(y)
          return _c

        lax.fori_loop(0, ycnt_s, row_body, None)

    return _

  lax.fori_loop(0, NATOMS // L, chunk_body, None)

  pltpu.async_copy(plane_v.at[pl.ds(0, 2 * GRID)],
                   out_h.at[wid], sem).wait()


def kernel(coordinates, active, occupancies, lmax, radial_densities,
           grid_to_cartesian):
  del lmax
  dtype = jnp.float32
  coords = coordinates[0].astype(dtype)
  ax = coords[:, 0]
  ay = coords[:, 1]
  az = coords[:, 2]
  occ = (occupancies[0] * active[0].astype(dtype)).astype(dtype)
  dens = radial_densities[0].astype(dtype) * occ[:, None]

  g = grid_to_cartesian.astype(dtype)
  rstep = jnp.asarray(RSTEP, dtype)
  gv = jnp.stack([
      g[0, 0], g[0, 1], g[0, 2], g[1, 1], g[1, 2], g[2, 2],
      -g[1, 2] / g[1, 1], 1.0 / jnp.abs(g[1, 1]), 1.0 / rstep,
      jnp.zeros((), dtype), jnp.zeros((), dtype), jnp.zeros((), dtype),
      jnp.zeros((), dtype), jnp.zeros((), dtype), jnp.zeros((), dtype),
      jnp.zeros((), dtype),
  ])
  pack = jnp.concatenate([ax, ay, az, gv])

  mesh = plsc.VectorSubcoreMesh(core_axis_name="c", subcore_axis_name="s")
  run = pl.kernel(
      _sc_body,
      out_type=jax.ShapeDtypeStruct((GRID, 2 * GRID, L), dtype),
      mesh=mesh,
      compiler_params=pltpu.CompilerParams(needs_layout_passes=False),
      scratch_types=[
          pltpu.VMEM((3 * NATOMS + L,), dtype),
          pltpu.VMEM((NATOMS, NRAD), dtype),
          pltpu.VMEM((2 * GRID + 4, L), dtype),
          pltpu.SemaphoreType.DMA,
      ],
  )
  out = run(pack, dens)
  return out.reshape((1, GRID, GRID, GRID))


# speedup vs baseline: 1.5469x; 1.0565x over previous
import jax
import jax.numpy as jnp
from jax import lax
from jax.experimental import pallas as pl
from jax.experimental.pallas import tpu as pltpu
from jax.experimental.pallas import tpu_sc as plsc

GRID = 32
RSTEP = 0.1
RMAX = 3.0
NATOMS = 128
NRAD = 64
L = 16


def _splat(vec, j):
  return jnp.full((L,), vec[j], dtype=vec.dtype)


def _sc_body(pack_h, dens_h, out_h, pack_v, dens_v, plane_v, sem):
  cid = lax.axis_index("c")
  sid = lax.axis_index("s")
  wid = sid * 2 + cid

  c1 = pltpu.async_copy(dens_h, dens_v, sem)
  c2 = pltpu.async_copy(pack_h, pack_v, sem)
  c1.wait()
  c2.wait()

  gv = pack_v[pl.ds(3 * NATOMS, L)]
  g00 = _splat(gv, 0)
  g01 = _splat(gv, 1)
  g02 = _splat(gv, 2)
  g11 = _splat(gv, 3)
  g12 = _splat(gv, 4)
  g22 = _splat(gv, 5)
  ngd = _splat(gv, 6)
  invg11 = _splat(gv, 7)
  inv_rstep = _splat(gv, 8)

  zf = jnp.full((L,), wid, dtype=jnp.int32).astype(jnp.float32)
  iota = lax.iota(jnp.int32, L)
  xf0 = iota.astype(jnp.float32)
  xf1 = (iota + 16).astype(jnp.float32)
  g00x0 = g00 * xf0
  g00x1 = g00 * xf1

  zero16 = jnp.zeros((L,), jnp.float32)

  def zero_body(r, _):
    plane_v[r] = zero16
    return _

  lax.fori_loop(0, 2 * GRID + 4, zero_body, None)

  rmax2 = jnp.full((L,), RMAX * RMAX, jnp.float32)
  rmax2_pad = jnp.full((L,), RMAX * RMAX + 1e-3, jnp.float32)
  half = jnp.full((L,), 0.5, jnp.float32)
  three_half = jnp.full((L,), 1.5, jnp.float32)
  magic = jnp.full((L,), 0x5F3759DF, jnp.int32)
  one_i = jnp.full((L,), 1, jnp.int32)
  zero_i = jnp.full((L,), 0, jnp.int32)

  def newton_rsqrt(a):
    bits = plsc.bitcast(a, jnp.int32)
    y0 = plsc.bitcast(magic - lax.shift_right_logical(bits, 1), jnp.float32)
    hx = half * a
    y0 = y0 * (three_half - hx * y0 * y0)
    y0 = y0 * (three_half - hx * y0 * y0)
    return y0

  def chunk_body(c, _):
    base = c * L
    axv = pack_v[pl.ds(base, L)]
    ayv = pack_v[pl.ds(NATOMS + base, L)]
    azv = pack_v[pl.ds(2 * NATOMS + base, L)]

    dzv = zf - azv
    cdzv = g22 * dzv
    remy = rmax2_pad - cdzv * cdzv
    ok = remy >= 0.0
    remy_nn = jnp.maximum(remy, 0.0)
    sy = remy_nn * newton_rsqrt(remy_nn)
    sy = sy * 1.00002 + 1e-3
    sy = jnp.where(ok, sy, -1.0)
    cyv = ngd * dzv
    hw = sy * invg11
    ylo_f = jnp.maximum(ayv + cyv - hw, 0.0)
    yhi_f = jnp.minimum(ayv + cyv + hw, GRID - 1.0)
    ilo = ylo_f.astype(jnp.int32)
    ylov = ilo + jnp.where(ilo.astype(jnp.float32) < ylo_f, one_i, zero_i)
    ycntv = yhi_f.astype(jnp.int32) - ylov + 1

    for j in range(L):
      ycnt_s = ycntv[j]

      @pl.when(ycnt_s > 0)
      def _():
        n = base + j
        nv = jnp.full((L,), n, jnp.int32)
        axs = _splat(axv, j)
        ays = _splat(ayv, j)
        dzs = _splat(dzv, j)
        cdzs = _splat(cdzv, j)
        ylo_s = ylov[j]
        cdz2 = cdzs * cdzs
        g12dz = g12 * dzs
        g02dz = g02 * dzs
        g00ax = g00 * axs

        @plsc.parallel_loop(0, ycnt_s, unroll=1)
        def row_body(yi):
          y = ylo_s + yi

          def one_row(yy):
            dyv = jnp.full((L,), yy, jnp.int32).astype(jnp.float32) - ays
            cdy = g12dz + g11 * dyv
            cyz2 = cdz2 + cdy * cdy
            rowbase = (g02dz + g01 * dyv) - g00ax
            r = yy * 2

            def do_half(hh, g00xf):
              cdx = rowbase + g00xf
              d2 = cdx * cdx + cyz2
              m = d2 <= rmax2
              y0 = newton_rsqrt(d2)
              dist = d2 * y0
              rad = dist * inv_rstep
              il_raw = rad.astype(jnp.int32)
              wh = rad - il_raw.astype(jnp.float32)
              il = jnp.minimum(il_raw, NRAD - 1)
              ih = jnp.minimum(il_raw + 1, NRAD - 1)
              dl = plsc.load_gather(dens_v, [nv, il])
              dh = plsc.load_gather(dens_v, [nv, ih])
              dens = dl + wh * (dh - dl)
              contrib = jnp.where(m, dens, 0.0)
              plsc.addupdate(plane_v.at[r + hh], contrib)

            do_half(0, g00x0)
            do_half(1, g00x1)

          one_row(y)

    return _

  lax.fori_loop(0, NATOMS // L, chunk_body, None)

  pltpu.async_copy(plane_v.at[pl.ds(0, 2 * GRID)],
                   out_h.at[wid], sem).wait()


def kernel(coordinates, active, occupancies, lmax, radial_densities,
           grid_to_cartesian):
  del lmax
  dtype = jnp.float32
  coords = coordinates[0].astype(dtype)
  ax = coords[:, 0]
  ay = coords[:, 1]
  az = coords[:, 2]
  occ = (occupancies[0] * active[0].astype(dtype)).astype(dtype)
  dens = radial_densities[0].astype(dtype) * occ[:, None]

  g = grid_to_cartesian.astype(dtype)
  rstep = jnp.asarray(RSTEP, dtype)
  gv = jnp.stack([
      g[0, 0], g[0, 1], g[0, 2], g[1, 1], g[1, 2], g[2, 2],
      -g[1, 2] / g[1, 1], 1.0 / jnp.abs(g[1, 1]), 1.0 / rstep,
      jnp.zeros((), dtype), jnp.zeros((), dtype), jnp.zeros((), dtype),
      jnp.zeros((), dtype), jnp.zeros((), dtype), jnp.zeros((), dtype),
      jnp.zeros((), dtype),
  ])
  pack = jnp.concatenate([ax, ay, az, gv])

  mesh = plsc.VectorSubcoreMesh(core_axis_name="c", subcore_axis_name="s")
  run = pl.kernel(
      _sc_body,
      out_type=jax.ShapeDtypeStruct((GRID, 2 * GRID, L), dtype),
      mesh=mesh,
      compiler_params=pltpu.CompilerParams(needs_layout_passes=False),
      scratch_types=[
          pltpu.VMEM((3 * NATOMS + L,), dtype),
          pltpu.VMEM((NATOMS, NRAD), dtype),
          pltpu.VMEM((2 * GRID + 4, L), dtype),
          pltpu.SemaphoreType.DMA,
      ],
  )
  out = run(pack, dens)
  return out.reshape((1, GRID, GRID, GRID))
